# baseline (speedup 1.0000x reference)
from functools import partial

import jax
import jax.numpy as jnp
from jax.experimental import pallas as pl
from jax.experimental.pallas import tpu as pltpu


def _wavefront_kernel(x_ref, w0_ref, w1_ref, w2_ref, b_ref, wlin_ref,
                      blin_ref, out_ref, h0, c0, h1, c1, h2, c2,
                      *, hidden, total_frames, d_pad):
    H, T, Dp = hidden, total_frames, d_pad
    f32 = jnp.float32

    def x_at(s):
        return x_ref[:, pl.ds(pl.multiple_of(s * Dp, Dp), Dp)]

    b0 = b_ref[0:1, :]
    b1 = b_ref[1:2, :]
    b2 = b_ref[2:3, :]

    def layer_step(inp, w_ref, bias, h_r, c_r):
        cat = jnp.concatenate([inp, h_r[...]], axis=1)
        pre = jnp.dot(cat, w_ref[...], preferred_element_type=f32) + bias
        s3 = jnp.tanh(pre[:, :3 * H])
        g_g = jnp.tanh(pre[:, 3 * H:])
        s_i = s3[:, 0 * H:1 * H]
        s_f = s3[:, 1 * H:2 * H]
        s_o = s3[:, 2 * H:3 * H]
        c = c_r[...]
        c_new = 0.5 * ((c + s_f * c) + (g_g + s_i * g_g))
        h_new = ((1.0 + s_o) * jnp.tanh(c_new)).astype(jnp.bfloat16)
        h_r[...] = h_new
        c_r[...] = c_new

    for r in (h0, h1, h2, c0, c1, c2):
        r[...] = jnp.zeros_like(r)

    layer_step(x_at(0), w0_ref, b0, h0, c0)
    h0_prev = h0[...]
    layer_step(x_at(1), w0_ref, b0, h0, c0)
    layer_step(h0_prev, w1_ref, b1, h1, c1)

    def body(s, carry):
        h0_prev = h0[...]
        h1_prev = h1[...]
        layer_step(x_at(s), w0_ref, b0, h0, c0)
        layer_step(h0_prev, w1_ref, b1, h1, c1)
        layer_step(h1_prev, w2_ref, b2, h2, c2)
        return carry

    jax.lax.fori_loop(2, T, body, 0, unroll=2)

    h0_prev = h0[...]
    h1_prev = h1[...]
    layer_step(h0_prev, w1_ref, b1, h1, c1)
    layer_step(h1_prev, w2_ref, b2, h2, c2)
    h1_prev = h1[...]
    layer_step(h1_prev, w2_ref, b2, h2, c2)

    y = jnp.dot(h2[...], wlin_ref[...], preferred_element_type=f32)
    y = jnp.maximum(y + blin_ref[...], 0.0)
    ssq = jnp.sum(y * y, axis=1, keepdims=True)
    out_ref[...] = y * jax.lax.rsqrt(jnp.maximum(ssq, 1e-12))


def kernel(utterances, w0x, whx, wh, b, wlin, blin):
    B, T, D_in = utterances.shape
    H = wh.shape[1]
    E = wlin.shape[1]
    L = b.shape[0]
    assert L == 3, "wavefront kernel is specialized to 3 LSTM layers"

    Dp = ((D_in + 127) // 128) * 128
    nb = 2
    Bb = -(-B // (8 * nb)) * 8
    B_pad = nb * Bb

    x = jnp.pad(utterances.astype(jnp.bfloat16),
                ((0, B_pad - B), (0, 0), (0, Dp - D_in)))
    x = x.reshape(B_pad, T * Dp)

    f32 = jnp.float32
    col = jnp.concatenate([jnp.full((3 * H,), 0.5, f32),
                           jnp.ones((H,), f32)])[None, :]
    w0xp = jnp.pad(w0x, ((0, Dp - D_in), (0, 0)))
    w0 = (jnp.concatenate([w0xp.astype(f32), 0.5 * wh[0].astype(f32)], 0)
          * col).astype(jnp.bfloat16)
    w1 = (jnp.concatenate([whx[0].astype(f32), wh[1].astype(f32)], 0)
          * (0.5 * col)).astype(jnp.bfloat16)
    w2 = (jnp.concatenate([whx[1].astype(f32), wh[2].astype(f32)], 0)
          * (0.5 * col)).astype(jnp.bfloat16)
    b_s = b * col
    wlin_s = (0.5 * wlin.astype(f32)).astype(jnp.bfloat16)

    kernel_fn = partial(_wavefront_kernel, hidden=H, total_frames=T,
                        d_pad=Dp)

    full = lambda bi: (0, 0)
    out = pl.pallas_call(
        kernel_fn,
        out_shape=jax.ShapeDtypeStruct((B_pad, E), jnp.float32),
        grid=(nb,),
        in_specs=[
            pl.BlockSpec((Bb, T * Dp), lambda bi: (bi, 0)),
            pl.BlockSpec((Dp + H, 4 * H), full),
            pl.BlockSpec((2 * H, 4 * H), full),
            pl.BlockSpec((2 * H, 4 * H), full),
            pl.BlockSpec((L, 4 * H), full),
            pl.BlockSpec((H, E), full),
            pl.BlockSpec((1, E), full),
        ],
        out_specs=pl.BlockSpec((Bb, E), lambda bi: (bi, 0)),
        scratch_shapes=[
            pltpu.VMEM((Bb, H), jnp.bfloat16),
            pltpu.VMEM((Bb, H), jnp.float32),
            pltpu.VMEM((Bb, H), jnp.bfloat16),
            pltpu.VMEM((Bb, H), jnp.float32),
            pltpu.VMEM((Bb, H), jnp.bfloat16),
            pltpu.VMEM((Bb, H), jnp.float32),
        ],
        compiler_params=pltpu.CompilerParams(
            dimension_semantics=("parallel",)),
    )(x, w0, w1, w2, b_s, wlin_s, blin)
    return out[:B]


# speedup vs baseline: 1.1661x; 1.1661x over previous
from functools import partial

import jax
import jax.numpy as jnp
from jax.experimental import pallas as pl
from jax.experimental.pallas import tpu as pltpu


def _wavefront_kernel(x_ref, w0_ref, w1_ref, w2_ref, b_ref, wlin_ref,
                      blin_ref, out_ref, h0, c0, h1, c1, h2, c2,
                      *, hidden, total_frames, d_pad):
    H, T, Dp = hidden, total_frames, d_pad
    f32 = jnp.float32

    def x_at(s):
        return x_ref[:, pl.ds(pl.multiple_of(s * Dp, Dp), Dp)]

    b0 = b_ref[0:1, :]
    b1 = b_ref[1:2, :]
    b2 = b_ref[2:3, :]

    def layer_step(inp, w_ref, bias, h_r, c_r):
        cat = jnp.concatenate([inp, h_r[...]], axis=1)
        pre = jnp.dot(cat, w_ref[...], preferred_element_type=f32) + bias
        s3 = jnp.tanh(pre[:, :3 * H])
        g_g = jnp.tanh(pre[:, 3 * H:])
        s_i = s3[:, 0 * H:1 * H]
        s_f = s3[:, 1 * H:2 * H]
        s_o = s3[:, 2 * H:3 * H]
        c = c_r[...]
        c_new = 0.5 * ((c + s_f * c) + (g_g + s_i * g_g))
        h_new = ((1.0 + s_o) * jnp.tanh(c_new)).astype(jnp.bfloat16)
        h_r[...] = h_new
        c_r[...] = c_new

    for r in (h0, h1, h2, c0, c1, c2):
        r[...] = jnp.zeros_like(r)

    layer_step(x_at(0), w0_ref, b0, h0, c0)
    h0_prev = h0[...]
    layer_step(x_at(1), w0_ref, b0, h0, c0)
    layer_step(h0_prev, w1_ref, b1, h1, c1)

    def body(s, carry):
        h0_prev = h0[...]
        h1_prev = h1[...]
        layer_step(x_at(s), w0_ref, b0, h0, c0)
        layer_step(h0_prev, w1_ref, b1, h1, c1)
        layer_step(h1_prev, w2_ref, b2, h2, c2)
        return carry

    jax.lax.fori_loop(2, T, body, 0, unroll=6)

    h0_prev = h0[...]
    h1_prev = h1[...]
    layer_step(h0_prev, w1_ref, b1, h1, c1)
    layer_step(h1_prev, w2_ref, b2, h2, c2)
    h1_prev = h1[...]
    layer_step(h1_prev, w2_ref, b2, h2, c2)

    y = jnp.dot(h2[...], wlin_ref[...], preferred_element_type=f32)
    y = jnp.maximum(y + blin_ref[...], 0.0)
    ssq = jnp.sum(y * y, axis=1, keepdims=True)
    out_ref[...] = y * jax.lax.rsqrt(jnp.maximum(ssq, 1e-12))


def kernel(utterances, w0x, whx, wh, b, wlin, blin):
    B, T, D_in = utterances.shape
    H = wh.shape[1]
    E = wlin.shape[1]
    L = b.shape[0]
    assert L == 3, "wavefront kernel is specialized to 3 LSTM layers"

    Dp = ((D_in + 127) // 128) * 128
    nb = 2
    Bb = -(-B // (8 * nb)) * 8
    B_pad = nb * Bb

    x = jnp.pad(utterances.astype(jnp.bfloat16),
                ((0, B_pad - B), (0, 0), (0, Dp - D_in)))
    x = x.reshape(B_pad, T * Dp)

    f32 = jnp.float32
    col = jnp.concatenate([jnp.full((3 * H,), 0.5, f32),
                           jnp.ones((H,), f32)])[None, :]
    w0xp = jnp.pad(w0x, ((0, Dp - D_in), (0, 0)))
    w0 = (jnp.concatenate([w0xp.astype(f32), 0.5 * wh[0].astype(f32)], 0)
          * col).astype(jnp.bfloat16)
    w1 = (jnp.concatenate([whx[0].astype(f32), wh[1].astype(f32)], 0)
          * (0.5 * col)).astype(jnp.bfloat16)
    w2 = (jnp.concatenate([whx[1].astype(f32), wh[2].astype(f32)], 0)
          * (0.5 * col)).astype(jnp.bfloat16)
    b_s = b * col
    wlin_s = (0.5 * wlin.astype(f32)).astype(jnp.bfloat16)

    kernel_fn = partial(_wavefront_kernel, hidden=H, total_frames=T,
                        d_pad=Dp)

    full = lambda bi: (0, 0)
    out = pl.pallas_call(
        kernel_fn,
        out_shape=jax.ShapeDtypeStruct((B_pad, E), jnp.float32),
        grid=(nb,),
        in_specs=[
            pl.BlockSpec((Bb, T * Dp), lambda bi: (bi, 0)),
            pl.BlockSpec((Dp + H, 4 * H), full),
            pl.BlockSpec((2 * H, 4 * H), full),
            pl.BlockSpec((2 * H, 4 * H), full),
            pl.BlockSpec((L, 4 * H), full),
            pl.BlockSpec((H, E), full),
            pl.BlockSpec((1, E), full),
        ],
        out_specs=pl.BlockSpec((Bb, E), lambda bi: (bi, 0)),
        scratch_shapes=[
            pltpu.VMEM((Bb, H), jnp.bfloat16),
            pltpu.VMEM((Bb, H), jnp.float32),
            pltpu.VMEM((Bb, H), jnp.bfloat16),
            pltpu.VMEM((Bb, H), jnp.float32),
            pltpu.VMEM((Bb, H), jnp.bfloat16),
            pltpu.VMEM((Bb, H), jnp.float32),
        ],
        compiler_params=pltpu.CompilerParams(
            dimension_semantics=("parallel",)),
    )(x, w0, w1, w2, b_s, wlin_s, blin)
    return out[:B]


# speedup vs baseline: 1.1928x; 1.0229x over previous
from functools import partial

import jax
import jax.numpy as jnp
from jax.experimental import pallas as pl
from jax.experimental.pallas import tpu as pltpu


def _wavefront_kernel(x_ref, w0_ref, w1_ref, w2_ref, b_ref, wlin_ref,
                      blin_ref, out_ref, h0, c0, h1, c1, h2, c2,
                      *, hidden, total_frames, d_pad):
    H, T, Dp = hidden, total_frames, d_pad
    f32 = jnp.float32

    def x_at(s):
        return x_ref[:, pl.ds(pl.multiple_of(s * Dp, Dp), Dp)]

    b0 = b_ref[0:1, :]
    b1 = b_ref[1:2, :]
    b2 = b_ref[2:3, :]

    def layer_step(inp, w_ref, bias, h_r, c_r):
        cat = jnp.concatenate([inp, h_r[...]], axis=1)
        pre = jnp.dot(cat, w_ref[...], preferred_element_type=f32) + bias
        s3 = jnp.tanh(pre[:, :3 * H])
        g_g = jnp.tanh(pre[:, 3 * H:])
        s_i = s3[:, 0 * H:1 * H]
        s_f = s3[:, 1 * H:2 * H]
        s_o = s3[:, 2 * H:3 * H]
        c = c_r[...]
        c_new = 0.5 * ((c + s_f * c) + (g_g + s_i * g_g))
        h_new = ((1.0 + s_o) * jnp.tanh(c_new)).astype(jnp.bfloat16)
        h_r[...] = h_new
        c_r[...] = c_new

    for r in (h0, h1, h2, c0, c1, c2):
        r[...] = jnp.zeros_like(r)

    layer_step(x_at(0), w0_ref, b0, h0, c0)
    h0_prev = h0[...]
    layer_step(x_at(1), w0_ref, b0, h0, c0)
    layer_step(h0_prev, w1_ref, b1, h1, c1)

    def body(s, carry):
        h0_prev = h0[...]
        h1_prev = h1[...]
        layer_step(x_at(s), w0_ref, b0, h0, c0)
        layer_step(h0_prev, w1_ref, b1, h1, c1)
        layer_step(h1_prev, w2_ref, b2, h2, c2)
        return carry

    jax.lax.fori_loop(2, T, body, 0, unroll=8)

    h0_prev = h0[...]
    h1_prev = h1[...]
    layer_step(h0_prev, w1_ref, b1, h1, c1)
    layer_step(h1_prev, w2_ref, b2, h2, c2)
    h1_prev = h1[...]
    layer_step(h1_prev, w2_ref, b2, h2, c2)

    y = jnp.dot(h2[...], wlin_ref[...], preferred_element_type=f32)
    y = jnp.maximum(y + blin_ref[...], 0.0)
    ssq = jnp.sum(y * y, axis=1, keepdims=True)
    out_ref[...] = y * jax.lax.rsqrt(jnp.maximum(ssq, 1e-12))


def kernel(utterances, w0x, whx, wh, b, wlin, blin):
    B, T, D_in = utterances.shape
    H = wh.shape[1]
    E = wlin.shape[1]
    L = b.shape[0]
    assert L == 3, "wavefront kernel is specialized to 3 LSTM layers"

    Dp = ((D_in + 127) // 128) * 128
    nb = 2
    Bb = -(-B // (8 * nb)) * 8
    B_pad = nb * Bb

    x = jnp.pad(utterances.astype(jnp.bfloat16),
                ((0, B_pad - B), (0, 0), (0, Dp - D_in)))
    x = x.reshape(B_pad, T * Dp)

    f32 = jnp.float32
    col = jnp.concatenate([jnp.full((3 * H,), 0.5, f32),
                           jnp.ones((H,), f32)])[None, :]
    w0xp = jnp.pad(w0x, ((0, Dp - D_in), (0, 0)))
    w0 = (jnp.concatenate([w0xp.astype(f32), 0.5 * wh[0].astype(f32)], 0)
          * col).astype(jnp.bfloat16)
    w1 = (jnp.concatenate([whx[0].astype(f32), wh[1].astype(f32)], 0)
          * (0.5 * col)).astype(jnp.bfloat16)
    w2 = (jnp.concatenate([whx[1].astype(f32), wh[2].astype(f32)], 0)
          * (0.5 * col)).astype(jnp.bfloat16)
    b_s = b * col
    wlin_s = (0.5 * wlin.astype(f32)).astype(jnp.bfloat16)

    kernel_fn = partial(_wavefront_kernel, hidden=H, total_frames=T,
                        d_pad=Dp)

    full = lambda bi: (0, 0)
    out = pl.pallas_call(
        kernel_fn,
        out_shape=jax.ShapeDtypeStruct((B_pad, E), jnp.float32),
        grid=(nb,),
        in_specs=[
            pl.BlockSpec((Bb, T * Dp), lambda bi: (bi, 0)),
            pl.BlockSpec((Dp + H, 4 * H), full),
            pl.BlockSpec((2 * H, 4 * H), full),
            pl.BlockSpec((2 * H, 4 * H), full),
            pl.BlockSpec((L, 4 * H), full),
            pl.BlockSpec((H, E), full),
            pl.BlockSpec((1, E), full),
        ],
        out_specs=pl.BlockSpec((Bb, E), lambda bi: (bi, 0)),
        scratch_shapes=[
            pltpu.VMEM((Bb, H), jnp.bfloat16),
            pltpu.VMEM((Bb, H), jnp.float32),
            pltpu.VMEM((Bb, H), jnp.bfloat16),
            pltpu.VMEM((Bb, H), jnp.float32),
            pltpu.VMEM((Bb, H), jnp.bfloat16),
            pltpu.VMEM((Bb, H), jnp.float32),
        ],
        compiler_params=pltpu.CompilerParams(
            dimension_semantics=("parallel",)),
    )(x, w0, w1, w2, b_s, wlin_s, blin)
    return out[:B]


# speedup vs baseline: 1.2125x; 1.0165x over previous
from functools import partial

import jax
import jax.numpy as jnp
from jax.experimental import pallas as pl
from jax.experimental.pallas import tpu as pltpu


def _wavefront_kernel(x_ref, w0_ref, w1_ref, w2_ref, b_ref, wlin_ref,
                      blin_ref, out_ref, h0, c0, h1, c1, h2, c2,
                      *, hidden, total_frames, d_pad):
    H, T, Dp = hidden, total_frames, d_pad
    f32 = jnp.float32

    def x_at(s):
        return x_ref[:, pl.ds(pl.multiple_of(s * Dp, Dp), Dp)]

    b0 = b_ref[0:1, :]
    b1 = b_ref[1:2, :]
    b2 = b_ref[2:3, :]

    def layer_step(inp, w_ref, bias, h_r, c_r):
        cat = jnp.concatenate([inp, h_r[...]], axis=1)
        pre = jnp.dot(cat, w_ref[...], preferred_element_type=f32) + bias
        s3 = jnp.tanh(pre[:, :3 * H])
        g_g = jnp.tanh(pre[:, 3 * H:])
        s_i = s3[:, 0 * H:1 * H]
        s_f = s3[:, 1 * H:2 * H]
        s_o = s3[:, 2 * H:3 * H]
        c = c_r[...]
        c_new = 0.5 * ((c + s_f * c) + (g_g + s_i * g_g))
        h_new = ((1.0 + s_o) * jnp.tanh(c_new)).astype(jnp.bfloat16)
        h_r[...] = h_new
        c_r[...] = c_new

    for r in (h0, h1, h2, c0, c1, c2):
        r[...] = jnp.zeros_like(r)

    layer_step(x_at(0), w0_ref, b0, h0, c0)
    h0_prev = h0[...]
    layer_step(x_at(1), w0_ref, b0, h0, c0)
    layer_step(h0_prev, w1_ref, b1, h1, c1)

    def body(s, carry):
        h0_prev = h0[...]
        h1_prev = h1[...]
        layer_step(x_at(s), w0_ref, b0, h0, c0)
        layer_step(h0_prev, w1_ref, b1, h1, c1)
        layer_step(h1_prev, w2_ref, b2, h2, c2)
        return carry

    jax.lax.fori_loop(2, T, body, 0, unroll=12)

    h0_prev = h0[...]
    h1_prev = h1[...]
    layer_step(h0_prev, w1_ref, b1, h1, c1)
    layer_step(h1_prev, w2_ref, b2, h2, c2)
    h1_prev = h1[...]
    layer_step(h1_prev, w2_ref, b2, h2, c2)

    y = jnp.dot(h2[...], wlin_ref[...], preferred_element_type=f32)
    y = jnp.maximum(y + blin_ref[...], 0.0)
    ssq = jnp.sum(y * y, axis=1, keepdims=True)
    out_ref[...] = y * jax.lax.rsqrt(jnp.maximum(ssq, 1e-12))


def kernel(utterances, w0x, whx, wh, b, wlin, blin):
    B, T, D_in = utterances.shape
    H = wh.shape[1]
    E = wlin.shape[1]
    L = b.shape[0]
    assert L == 3, "wavefront kernel is specialized to 3 LSTM layers"

    Dp = ((D_in + 127) // 128) * 128
    nb = 2
    Bb = -(-B // (8 * nb)) * 8
    B_pad = nb * Bb

    x = jnp.pad(utterances.astype(jnp.bfloat16),
                ((0, B_pad - B), (0, 0), (0, Dp - D_in)))
    x = x.reshape(B_pad, T * Dp)

    f32 = jnp.float32
    col = jnp.concatenate([jnp.full((3 * H,), 0.5, f32),
                           jnp.ones((H,), f32)])[None, :]
    w0xp = jnp.pad(w0x, ((0, Dp - D_in), (0, 0)))
    w0 = (jnp.concatenate([w0xp.astype(f32), 0.5 * wh[0].astype(f32)], 0)
          * col).astype(jnp.bfloat16)
    w1 = (jnp.concatenate([whx[0].astype(f32), wh[1].astype(f32)], 0)
          * (0.5 * col)).astype(jnp.bfloat16)
    w2 = (jnp.concatenate([whx[1].astype(f32), wh[2].astype(f32)], 0)
          * (0.5 * col)).astype(jnp.bfloat16)
    b_s = b * col
    wlin_s = (0.5 * wlin.astype(f32)).astype(jnp.bfloat16)

    kernel_fn = partial(_wavefront_kernel, hidden=H, total_frames=T,
                        d_pad=Dp)

    full = lambda bi: (0, 0)
    out = pl.pallas_call(
        kernel_fn,
        out_shape=jax.ShapeDtypeStruct((B_pad, E), jnp.float32),
        grid=(nb,),
        in_specs=[
            pl.BlockSpec((Bb, T * Dp), lambda bi: (bi, 0)),
            pl.BlockSpec((Dp + H, 4 * H), full),
            pl.BlockSpec((2 * H, 4 * H), full),
            pl.BlockSpec((2 * H, 4 * H), full),
            pl.BlockSpec((L, 4 * H), full),
            pl.BlockSpec((H, E), full),
            pl.BlockSpec((1, E), full),
        ],
        out_specs=pl.BlockSpec((Bb, E), lambda bi: (bi, 0)),
        scratch_shapes=[
            pltpu.VMEM((Bb, H), jnp.bfloat16),
            pltpu.VMEM((Bb, H), jnp.float32),
            pltpu.VMEM((Bb, H), jnp.bfloat16),
            pltpu.VMEM((Bb, H), jnp.float32),
            pltpu.VMEM((Bb, H), jnp.bfloat16),
            pltpu.VMEM((Bb, H), jnp.float32),
        ],
        compiler_params=pltpu.CompilerParams(
            dimension_semantics=("parallel",)),
    )(x, w0, w1, w2, b_s, wlin_s, blin)
    return out[:B]


# speedup vs baseline: 1.2197x; 1.0059x over previous
from functools import partial

import jax
import jax.numpy as jnp
from jax.experimental import pallas as pl
from jax.experimental.pallas import tpu as pltpu


def _wavefront_kernel(x_ref, w0_ref, w1_ref, w2_ref, b_ref, wlin_ref,
                      blin_ref, out_ref, h0, c0, h1, c1, h2, c2,
                      *, hidden, total_frames, d_pad):
    H, T, Dp = hidden, total_frames, d_pad
    f32 = jnp.float32

    def x_at(s):
        return x_ref[:, pl.ds(pl.multiple_of(s * Dp, Dp), Dp)]

    b0 = b_ref[0:1, :]
    b1 = b_ref[1:2, :]
    b2 = b_ref[2:3, :]

    def layer_step(inp, w_ref, bias, h_r, c_r):
        cat = jnp.concatenate([inp, h_r[...]], axis=1)
        pre = jnp.dot(cat, w_ref[...], preferred_element_type=f32) + bias
        s3 = jnp.tanh(pre[:, :3 * H])
        g_g = jnp.tanh(pre[:, 3 * H:])
        s_i = s3[:, 0 * H:1 * H]
        s_f = s3[:, 1 * H:2 * H]
        s_o = s3[:, 2 * H:3 * H]
        c = c_r[...]
        c_new = 0.5 * ((c + s_f * c) + (g_g + s_i * g_g))
        h_new = ((1.0 + s_o) * jnp.tanh(c_new)).astype(jnp.bfloat16)
        h_r[...] = h_new
        c_r[...] = c_new

    for r in (h0, h1, h2, c0, c1, c2):
        r[...] = jnp.zeros_like(r)

    layer_step(x_at(0), w0_ref, b0, h0, c0)
    h0_prev = h0[...]
    layer_step(x_at(1), w0_ref, b0, h0, c0)
    layer_step(h0_prev, w1_ref, b1, h1, c1)

    def body(s, carry):
        h0_prev = h0[...]
        h1_prev = h1[...]
        layer_step(x_at(s), w0_ref, b0, h0, c0)
        layer_step(h0_prev, w1_ref, b1, h1, c1)
        layer_step(h1_prev, w2_ref, b2, h2, c2)
        return carry

    jax.lax.fori_loop(2, T, body, 0, unroll=16)

    h0_prev = h0[...]
    h1_prev = h1[...]
    layer_step(h0_prev, w1_ref, b1, h1, c1)
    layer_step(h1_prev, w2_ref, b2, h2, c2)
    h1_prev = h1[...]
    layer_step(h1_prev, w2_ref, b2, h2, c2)

    y = jnp.dot(h2[...], wlin_ref[...], preferred_element_type=f32)
    y = jnp.maximum(y + blin_ref[...], 0.0)
    ssq = jnp.sum(y * y, axis=1, keepdims=True)
    out_ref[...] = y * jax.lax.rsqrt(jnp.maximum(ssq, 1e-12))


def kernel(utterances, w0x, whx, wh, b, wlin, blin):
    B, T, D_in = utterances.shape
    H = wh.shape[1]
    E = wlin.shape[1]
    L = b.shape[0]
    assert L == 3, "wavefront kernel is specialized to 3 LSTM layers"

    Dp = ((D_in + 127) // 128) * 128
    nb = 2
    Bb = -(-B // (8 * nb)) * 8
    B_pad = nb * Bb

    x = jnp.pad(utterances.astype(jnp.bfloat16),
                ((0, B_pad - B), (0, 0), (0, Dp - D_in)))
    x = x.reshape(B_pad, T * Dp)

    f32 = jnp.float32
    col = jnp.concatenate([jnp.full((3 * H,), 0.5, f32),
                           jnp.ones((H,), f32)])[None, :]
    w0xp = jnp.pad(w0x, ((0, Dp - D_in), (0, 0)))
    w0 = (jnp.concatenate([w0xp.astype(f32), 0.5 * wh[0].astype(f32)], 0)
          * col).astype(jnp.bfloat16)
    w1 = (jnp.concatenate([whx[0].astype(f32), wh[1].astype(f32)], 0)
          * (0.5 * col)).astype(jnp.bfloat16)
    w2 = (jnp.concatenate([whx[1].astype(f32), wh[2].astype(f32)], 0)
          * (0.5 * col)).astype(jnp.bfloat16)
    b_s = b * col
    wlin_s = (0.5 * wlin.astype(f32)).astype(jnp.bfloat16)

    kernel_fn = partial(_wavefront_kernel, hidden=H, total_frames=T,
                        d_pad=Dp)

    full = lambda bi: (0, 0)
    out = pl.pallas_call(
        kernel_fn,
        out_shape=jax.ShapeDtypeStruct((B_pad, E), jnp.float32),
        grid=(nb,),
        in_specs=[
            pl.BlockSpec((Bb, T * Dp), lambda bi: (bi, 0)),
            pl.BlockSpec((Dp + H, 4 * H), full),
            pl.BlockSpec((2 * H, 4 * H), full),
            pl.BlockSpec((2 * H, 4 * H), full),
            pl.BlockSpec((L, 4 * H), full),
            pl.BlockSpec((H, E), full),
            pl.BlockSpec((1, E), full),
        ],
        out_specs=pl.BlockSpec((Bb, E), lambda bi: (bi, 0)),
        scratch_shapes=[
            pltpu.VMEM((Bb, H), jnp.bfloat16),
            pltpu.VMEM((Bb, H), jnp.float32),
            pltpu.VMEM((Bb, H), jnp.bfloat16),
            pltpu.VMEM((Bb, H), jnp.float32),
            pltpu.VMEM((Bb, H), jnp.bfloat16),
            pltpu.VMEM((Bb, H), jnp.float32),
        ],
        compiler_params=pltpu.CompilerParams(
            dimension_semantics=("parallel",)),
    )(x, w0, w1, w2, b_s, wlin_s, blin)
    return out[:B]


# speedup vs baseline: 1.2296x; 1.0081x over previous
from functools import partial

import jax
import jax.numpy as jnp
from jax.experimental import pallas as pl
from jax.experimental.pallas import tpu as pltpu


def _wavefront_kernel(x_ref, w0_ref, w1_ref, w2_ref, b_ref, wlin_ref,
                      blin_ref, out_ref, h0, c0, h1, c1, h2, c2,
                      *, hidden, total_frames, d_pad):
    H, T, Dp = hidden, total_frames, d_pad
    f32 = jnp.float32

    def x_at(s):
        return x_ref[:, pl.ds(pl.multiple_of(s * Dp, Dp), Dp)]

    b0 = b_ref[0:1, :]
    b1 = b_ref[1:2, :]
    b2 = b_ref[2:3, :]

    def layer_step(inp, w_ref, bias, h_r, c_r):
        cat = jnp.concatenate([inp, h_r[...]], axis=1)
        pre = jnp.dot(cat, w_ref[...], preferred_element_type=f32) + bias
        s3 = jnp.tanh(pre[:, :3 * H])
        g_g = jnp.tanh(pre[:, 3 * H:])
        s_i = s3[:, 0 * H:1 * H]
        s_f = s3[:, 1 * H:2 * H]
        s_o = s3[:, 2 * H:3 * H]
        c = c_r[...]
        c_new = 0.5 * ((c + s_f * c) + (g_g + s_i * g_g))
        h_new = ((1.0 + s_o) * jnp.tanh(c_new)).astype(jnp.bfloat16)
        h_r[...] = h_new
        c_r[...] = c_new

    for r in (h0, h1, h2, c0, c1, c2):
        r[...] = jnp.zeros_like(r)

    layer_step(x_at(0), w0_ref, b0, h0, c0)
    h0_prev = h0[...]
    layer_step(x_at(1), w0_ref, b0, h0, c0)
    layer_step(h0_prev, w1_ref, b1, h1, c1)

    def body(s, carry):
        h0_prev = h0[...]
        h1_prev = h1[...]
        layer_step(x_at(s), w0_ref, b0, h0, c0)
        layer_step(h0_prev, w1_ref, b1, h1, c1)
        layer_step(h1_prev, w2_ref, b2, h2, c2)
        return carry

    jax.lax.fori_loop(2, T, body, 0, unroll=24)

    h0_prev = h0[...]
    h1_prev = h1[...]
    layer_step(h0_prev, w1_ref, b1, h1, c1)
    layer_step(h1_prev, w2_ref, b2, h2, c2)
    h1_prev = h1[...]
    layer_step(h1_prev, w2_ref, b2, h2, c2)

    y = jnp.dot(h2[...], wlin_ref[...], preferred_element_type=f32)
    y = jnp.maximum(y + blin_ref[...], 0.0)
    ssq = jnp.sum(y * y, axis=1, keepdims=True)
    out_ref[...] = y * jax.lax.rsqrt(jnp.maximum(ssq, 1e-12))


def kernel(utterances, w0x, whx, wh, b, wlin, blin):
    B, T, D_in = utterances.shape
    H = wh.shape[1]
    E = wlin.shape[1]
    L = b.shape[0]
    assert L == 3, "wavefront kernel is specialized to 3 LSTM layers"

    Dp = ((D_in + 127) // 128) * 128
    nb = 2
    Bb = -(-B // (8 * nb)) * 8
    B_pad = nb * Bb

    x = jnp.pad(utterances.astype(jnp.bfloat16),
                ((0, B_pad - B), (0, 0), (0, Dp - D_in)))
    x = x.reshape(B_pad, T * Dp)

    f32 = jnp.float32
    col = jnp.concatenate([jnp.full((3 * H,), 0.5, f32),
                           jnp.ones((H,), f32)])[None, :]
    w0xp = jnp.pad(w0x, ((0, Dp - D_in), (0, 0)))
    w0 = (jnp.concatenate([w0xp.astype(f32), 0.5 * wh[0].astype(f32)], 0)
          * col).astype(jnp.bfloat16)
    w1 = (jnp.concatenate([whx[0].astype(f32), wh[1].astype(f32)], 0)
          * (0.5 * col)).astype(jnp.bfloat16)
    w2 = (jnp.concatenate([whx[1].astype(f32), wh[2].astype(f32)], 0)
          * (0.5 * col)).astype(jnp.bfloat16)
    b_s = b * col
    wlin_s = (0.5 * wlin.astype(f32)).astype(jnp.bfloat16)

    kernel_fn = partial(_wavefront_kernel, hidden=H, total_frames=T,
                        d_pad=Dp)

    full = lambda bi: (0, 0)
    out = pl.pallas_call(
        kernel_fn,
        out_shape=jax.ShapeDtypeStruct((B_pad, E), jnp.float32),
        grid=(nb,),
        in_specs=[
            pl.BlockSpec((Bb, T * Dp), lambda bi: (bi, 0)),
            pl.BlockSpec((Dp + H, 4 * H), full),
            pl.BlockSpec((2 * H, 4 * H), full),
            pl.BlockSpec((2 * H, 4 * H), full),
            pl.BlockSpec((L, 4 * H), full),
            pl.BlockSpec((H, E), full),
            pl.BlockSpec((1, E), full),
        ],
        out_specs=pl.BlockSpec((Bb, E), lambda bi: (bi, 0)),
        scratch_shapes=[
            pltpu.VMEM((Bb, H), jnp.bfloat16),
            pltpu.VMEM((Bb, H), jnp.float32),
            pltpu.VMEM((Bb, H), jnp.bfloat16),
            pltpu.VMEM((Bb, H), jnp.float32),
            pltpu.VMEM((Bb, H), jnp.bfloat16),
            pltpu.VMEM((Bb, H), jnp.float32),
        ],
        compiler_params=pltpu.CompilerParams(
            dimension_semantics=("parallel",)),
    )(x, w0, w1, w2, b_s, wlin_s, blin)
    return out[:B]


# speedup vs baseline: 1.2506x; 1.0171x over previous
from functools import partial

import jax
import jax.numpy as jnp
from jax.experimental import pallas as pl
from jax.experimental.pallas import tpu as pltpu


def _wavefront_kernel(x_ref, w0_ref, w1_ref, w2_ref, b_ref, wlin_ref,
                      blin_ref, out_ref, h0, c0, h1, c1, h2, c2,
                      *, hidden, total_frames, d_pad):
    H, T, Dp = hidden, total_frames, d_pad
    f32 = jnp.float32

    def x_at(s):
        return x_ref[:, pl.ds(pl.multiple_of(s * Dp, Dp), Dp)]

    b0 = b_ref[0:1, :]
    b1 = b_ref[1:2, :]
    b2 = b_ref[2:3, :]

    def layer_step(inp, w_ref, bias, h_r, c_r):
        cat = jnp.concatenate([inp, h_r[...]], axis=1)
        pre = jnp.dot(cat, w_ref[...], preferred_element_type=f32) + bias
        s3 = jnp.tanh(pre[:, :3 * H])
        g_g = jnp.tanh(pre[:, 3 * H:])
        s_i = s3[:, 0 * H:1 * H]
        s_f = s3[:, 1 * H:2 * H]
        s_o = s3[:, 2 * H:3 * H]
        c = c_r[...]
        c_new = 0.5 * ((c + s_f * c) + (g_g + s_i * g_g))
        h_new = ((1.0 + s_o) * jnp.tanh(c_new)).astype(jnp.bfloat16)
        h_r[...] = h_new
        c_r[...] = c_new

    for r in (h0, h1, h2, c0, c1, c2):
        r[...] = jnp.zeros_like(r)

    layer_step(x_at(0), w0_ref, b0, h0, c0)
    h0_prev = h0[...]
    layer_step(x_at(1), w0_ref, b0, h0, c0)
    layer_step(h0_prev, w1_ref, b1, h1, c1)

    def body(s, carry):
        h0_prev = h0[...]
        h1_prev = h1[...]
        layer_step(x_at(s), w0_ref, b0, h0, c0)
        layer_step(h0_prev, w1_ref, b1, h1, c1)
        layer_step(h1_prev, w2_ref, b2, h2, c2)
        return carry

    jax.lax.fori_loop(2, T, body, 0, unroll=32)

    h0_prev = h0[...]
    h1_prev = h1[...]
    layer_step(h0_prev, w1_ref, b1, h1, c1)
    layer_step(h1_prev, w2_ref, b2, h2, c2)
    h1_prev = h1[...]
    layer_step(h1_prev, w2_ref, b2, h2, c2)

    y = jnp.dot(h2[...], wlin_ref[...], preferred_element_type=f32)
    y = jnp.maximum(y + blin_ref[...], 0.0)
    ssq = jnp.sum(y * y, axis=1, keepdims=True)
    out_ref[...] = y * jax.lax.rsqrt(jnp.maximum(ssq, 1e-12))


def kernel(utterances, w0x, whx, wh, b, wlin, blin):
    B, T, D_in = utterances.shape
    H = wh.shape[1]
    E = wlin.shape[1]
    L = b.shape[0]
    assert L == 3, "wavefront kernel is specialized to 3 LSTM layers"

    Dp = ((D_in + 127) // 128) * 128
    nb = 2
    Bb = -(-B // (8 * nb)) * 8
    B_pad = nb * Bb

    x = jnp.pad(utterances.astype(jnp.bfloat16),
                ((0, B_pad - B), (0, 0), (0, Dp - D_in)))
    x = x.reshape(B_pad, T * Dp)

    f32 = jnp.float32
    col = jnp.concatenate([jnp.full((3 * H,), 0.5, f32),
                           jnp.ones((H,), f32)])[None, :]
    w0xp = jnp.pad(w0x, ((0, Dp - D_in), (0, 0)))
    w0 = (jnp.concatenate([w0xp.astype(f32), 0.5 * wh[0].astype(f32)], 0)
          * col).astype(jnp.bfloat16)
    w1 = (jnp.concatenate([whx[0].astype(f32), wh[1].astype(f32)], 0)
          * (0.5 * col)).astype(jnp.bfloat16)
    w2 = (jnp.concatenate([whx[1].astype(f32), wh[2].astype(f32)], 0)
          * (0.5 * col)).astype(jnp.bfloat16)
    b_s = b * col
    wlin_s = (0.5 * wlin.astype(f32)).astype(jnp.bfloat16)

    kernel_fn = partial(_wavefront_kernel, hidden=H, total_frames=T,
                        d_pad=Dp)

    full = lambda bi: (0, 0)
    out = pl.pallas_call(
        kernel_fn,
        out_shape=jax.ShapeDtypeStruct((B_pad, E), jnp.float32),
        grid=(nb,),
        in_specs=[
            pl.BlockSpec((Bb, T * Dp), lambda bi: (bi, 0)),
            pl.BlockSpec((Dp + H, 4 * H), full),
            pl.BlockSpec((2 * H, 4 * H), full),
            pl.BlockSpec((2 * H, 4 * H), full),
            pl.BlockSpec((L, 4 * H), full),
            pl.BlockSpec((H, E), full),
            pl.BlockSpec((1, E), full),
        ],
        out_specs=pl.BlockSpec((Bb, E), lambda bi: (bi, 0)),
        scratch_shapes=[
            pltpu.VMEM((Bb, H), jnp.bfloat16),
            pltpu.VMEM((Bb, H), jnp.float32),
            pltpu.VMEM((Bb, H), jnp.bfloat16),
            pltpu.VMEM((Bb, H), jnp.float32),
            pltpu.VMEM((Bb, H), jnp.bfloat16),
            pltpu.VMEM((Bb, H), jnp.float32),
        ],
        compiler_params=pltpu.CompilerParams(
            dimension_semantics=("parallel",)),
    )(x, w0, w1, w2, b_s, wlin_s, blin)
    return out[:B]


# speedup vs baseline: 1.2542x; 1.0029x over previous
from functools import partial

import jax
import jax.numpy as jnp
from jax.experimental import pallas as pl
from jax.experimental.pallas import tpu as pltpu


def _wavefront_kernel(x_ref, w0_ref, w1_ref, w2_ref, b_ref, wlin_ref,
                      blin_ref, out_ref, h0, c0, h1, c1, h2, c2,
                      *, hidden, total_frames, d_pad):
    H, T, Dp = hidden, total_frames, d_pad
    f32 = jnp.float32

    def x_at(s):
        return x_ref[:, pl.ds(pl.multiple_of(s * Dp, Dp), Dp)]

    b0 = b_ref[0:1, :]
    b1 = b_ref[1:2, :]
    b2 = b_ref[2:3, :]

    def layer_step(inp, w_ref, bias, h_r, c_r):
        cat = jnp.concatenate([inp, h_r[...]], axis=1)
        pre = jnp.dot(cat, w_ref[...], preferred_element_type=f32) + bias
        s3 = jnp.tanh(pre[:, :3 * H])
        g_g = jnp.tanh(pre[:, 3 * H:])
        s_i = s3[:, 0 * H:1 * H]
        s_f = s3[:, 1 * H:2 * H]
        s_o = s3[:, 2 * H:3 * H]
        c = c_r[...]
        c_new = 0.5 * ((c + s_f * c) + (g_g + s_i * g_g))
        h_new = ((1.0 + s_o) * jnp.tanh(c_new)).astype(jnp.bfloat16)
        h_r[...] = h_new
        c_r[...] = c_new

    for r in (h0, h1, h2, c0, c1, c2):
        r[...] = jnp.zeros_like(r)

    layer_step(x_at(0), w0_ref, b0, h0, c0)
    h0_prev = h0[...]
    layer_step(x_at(1), w0_ref, b0, h0, c0)
    layer_step(h0_prev, w1_ref, b1, h1, c1)

    def body(s, carry):
        h0_prev = h0[...]
        h1_prev = h1[...]
        layer_step(x_at(s), w0_ref, b0, h0, c0)
        layer_step(h0_prev, w1_ref, b1, h1, c1)
        layer_step(h1_prev, w2_ref, b2, h2, c2)
        return carry

    jax.lax.fori_loop(2, T, body, 0, unroll=48)

    h0_prev = h0[...]
    h1_prev = h1[...]
    layer_step(h0_prev, w1_ref, b1, h1, c1)
    layer_step(h1_prev, w2_ref, b2, h2, c2)
    h1_prev = h1[...]
    layer_step(h1_prev, w2_ref, b2, h2, c2)

    y = jnp.dot(h2[...], wlin_ref[...], preferred_element_type=f32)
    y = jnp.maximum(y + blin_ref[...], 0.0)
    ssq = jnp.sum(y * y, axis=1, keepdims=True)
    out_ref[...] = y * jax.lax.rsqrt(jnp.maximum(ssq, 1e-12))


def kernel(utterances, w0x, whx, wh, b, wlin, blin):
    B, T, D_in = utterances.shape
    H = wh.shape[1]
    E = wlin.shape[1]
    L = b.shape[0]
    assert L == 3, "wavefront kernel is specialized to 3 LSTM layers"

    Dp = ((D_in + 127) // 128) * 128
    nb = 2
    Bb = -(-B // (8 * nb)) * 8
    B_pad = nb * Bb

    x = jnp.pad(utterances.astype(jnp.bfloat16),
                ((0, B_pad - B), (0, 0), (0, Dp - D_in)))
    x = x.reshape(B_pad, T * Dp)

    f32 = jnp.float32
    col = jnp.concatenate([jnp.full((3 * H,), 0.5, f32),
                           jnp.ones((H,), f32)])[None, :]
    w0xp = jnp.pad(w0x, ((0, Dp - D_in), (0, 0)))
    w0 = (jnp.concatenate([w0xp.astype(f32), 0.5 * wh[0].astype(f32)], 0)
          * col).astype(jnp.bfloat16)
    w1 = (jnp.concatenate([whx[0].astype(f32), wh[1].astype(f32)], 0)
          * (0.5 * col)).astype(jnp.bfloat16)
    w2 = (jnp.concatenate([whx[1].astype(f32), wh[2].astype(f32)], 0)
          * (0.5 * col)).astype(jnp.bfloat16)
    b_s = b * col
    wlin_s = (0.5 * wlin.astype(f32)).astype(jnp.bfloat16)

    kernel_fn = partial(_wavefront_kernel, hidden=H, total_frames=T,
                        d_pad=Dp)

    full = lambda bi: (0, 0)
    out = pl.pallas_call(
        kernel_fn,
        out_shape=jax.ShapeDtypeStruct((B_pad, E), jnp.float32),
        grid=(nb,),
        in_specs=[
            pl.BlockSpec((Bb, T * Dp), lambda bi: (bi, 0)),
            pl.BlockSpec((Dp + H, 4 * H), full),
            pl.BlockSpec((2 * H, 4 * H), full),
            pl.BlockSpec((2 * H, 4 * H), full),
            pl.BlockSpec((L, 4 * H), full),
            pl.BlockSpec((H, E), full),
            pl.BlockSpec((1, E), full),
        ],
        out_specs=pl.BlockSpec((Bb, E), lambda bi: (bi, 0)),
        scratch_shapes=[
            pltpu.VMEM((Bb, H), jnp.bfloat16),
            pltpu.VMEM((Bb, H), jnp.float32),
            pltpu.VMEM((Bb, H), jnp.bfloat16),
            pltpu.VMEM((Bb, H), jnp.float32),
            pltpu.VMEM((Bb, H), jnp.bfloat16),
            pltpu.VMEM((Bb, H), jnp.float32),
        ],
        compiler_params=pltpu.CompilerParams(
            dimension_semantics=("parallel",)),
    )(x, w0, w1, w2, b_s, wlin_s, blin)
    return out[:B]
